# baseline (speedup 1.0000x reference)
import functools

import jax
import jax.numpy as jnp
from jax import lax
from jax.experimental import pallas as pl
from jax.experimental.pallas import tpu as pltpu
from jax.experimental.pallas import tpu_sc as plsc


def _make_gather(V, D, B):
    info = plsc.get_sparse_core_info()
    NC, NS = info.num_cores, info.num_subcores
    NW = NC * NS
    assert B % NW == 0
    b_per_w = B // NW
    C = 16
    NB = 7
    n_chunks = b_per_w // C
    assert b_per_w % C == 0

    mesh = plsc.VectorSubcoreMesh(core_axis_name="c", subcore_axis_name="s")

    @functools.partial(
        pl.kernel,
        out_type=jax.ShapeDtypeStruct((B, D), jnp.float32),
        mesh=mesh,
        scratch_types=[
            pltpu.VMEM((b_per_w,), jnp.int32),
            *[pltpu.VMEM((C, D), jnp.float32) for _ in range(NB)],
            *[pltpu.SemaphoreType.DMA for _ in range(2 * NB)],
        ],
    )
    def gather_kernel(table_hbm, idx_hbm, out_hbm, idx_v, *bufs_and_sems):
        bufs = bufs_and_sems[:NB]
        gsem = bufs_and_sems[NB : 2 * NB]
        wsem = bufs_and_sems[2 * NB :]
        wid = lax.axis_index("s") * NC + lax.axis_index("c")
        base = wid * b_per_w
        pltpu.sync_copy(idx_hbm.at[pl.ds(base, b_per_w)], idx_v)

        def gather(g):
            b = g % NB
            return pltpu.async_copy(
                table_hbm.at[idx_v.at[pl.ds(g * C, C)]], bufs[b], gsem[b]
            )

        rd = {g: gather(g) for g in range(NB - 1)}
        wr = {}
        for g in range(n_chunks):
            b = g % NB
            rd[g].wait()
            wr[g] = pltpu.async_copy(
                bufs[b], out_hbm.at[pl.ds(base + g * C, C)], wsem[b]
            )
            if g + NB - 1 < n_chunks:
                if g - 1 >= 0:
                    wr[g - 1].wait()
                rd[g + NB - 1] = gather(g + NB - 1)
        for g in range(n_chunks - NB + 1, n_chunks):
            wr[g - 1].wait()
        wr[n_chunks - 1].wait()

    return gather_kernel


def kernel(positional_encoding, t):
    V, D = positional_encoding.shape
    (B,) = t.shape
    gather = _make_gather(V, D, B)
    return gather(positional_encoding, t.astype(jnp.int32))


# speedup vs baseline: 1.0187x; 1.0187x over previous
import functools

import jax
import jax.numpy as jnp
from jax import lax
from jax.experimental import pallas as pl
from jax.experimental.pallas import tpu as pltpu
from jax.experimental.pallas import tpu_sc as plsc


def _make_gather(V, D, B):
    info = plsc.get_sparse_core_info()
    NC, NS = info.num_cores, info.num_subcores
    NW = NC * NS
    assert B % NW == 0
    b_per_w = B // NW
    C = 8
    NB = 8
    n_chunks = b_per_w // C
    assert n_chunks % NB == 0

    mesh = plsc.VectorSubcoreMesh(core_axis_name="c", subcore_axis_name="s")

    @functools.partial(
        pl.kernel,
        out_type=jax.ShapeDtypeStruct((B, D), jnp.float32),
        mesh=mesh,
        scratch_types=[
            pltpu.VMEM((b_per_w,), jnp.int32),
            *[pltpu.VMEM((C, D), jnp.float32) for _ in range(NB)],
            *[pltpu.SemaphoreType.DMA for _ in range(2 * NB)],
        ],
    )
    def gather_kernel(table_hbm, idx_hbm, out_hbm, idx_v, *bufs_and_sems):
        bufs = bufs_and_sems[:NB]
        gsem = bufs_and_sems[NB : 2 * NB]
        wsem = bufs_and_sems[2 * NB :]
        wid = lax.axis_index("s") * NC + lax.axis_index("c")
        base = wid * b_per_w
        pltpu.sync_copy(idx_hbm.at[pl.ds(base, b_per_w)], idx_v)

        def rd_issue(g, b):
            pltpu.async_copy(
                table_hbm.at[idx_v.at[pl.ds(g * C, C)]], bufs[b], gsem[b]
            )

        def rd_wait(b):
            pltpu.make_async_copy(
                table_hbm.at[pl.ds(0, C)], bufs[b], gsem[b]
            ).wait()

        def wr_issue(g, b):
            pltpu.async_copy(
                bufs[b], out_hbm.at[pl.ds(base + g * C, C)], wsem[b]
            )

        def wr_wait(b):
            pltpu.make_async_copy(
                bufs[b], out_hbm.at[pl.ds(base, C)], wsem[b]
            ).wait()

        for g in range(NB - 1):
            rd_issue(g, g)
        rd_wait(0)
        wr_issue(0, 0)
        rd_issue(NB - 1, NB - 1)
        for g in range(1, NB):
            rd_wait(g % NB)
            wr_issue(g, g % NB)
            wr_wait((g - 1) % NB)
            rd_issue(g + NB - 1, (g + NB - 1) % NB)

        @pl.loop(1, n_chunks // NB - 1)
        def _steady(k):
            for b in range(NB):
                g = NB * k + b
                rd_wait(b)
                wr_issue(g, b)
                wr_wait((b - 1) % NB)
                rd_issue(g + NB - 1, (b + NB - 1) % NB)

        g0 = n_chunks - NB
        rd_wait(g0 % NB)
        wr_issue(g0, g0 % NB)
        wr_wait((g0 - 1) % NB)
        rd_issue(n_chunks - 1, (n_chunks - 1) % NB)
        for g in range(g0 + 1, n_chunks):
            rd_wait(g % NB)
            wr_issue(g, g % NB)
            wr_wait((g - 1) % NB)
        wr_wait((n_chunks - 1) % NB)

    return gather_kernel


def kernel(positional_encoding, t):
    V, D = positional_encoding.shape
    (B,) = t.shape
    gather = _make_gather(V, D, B)
    return gather(positional_encoding, t.astype(jnp.int32))
